# initial kernel scaffold (unmeasured)
import jax
import jax.numpy as jnp
from jax import lax
from jax.experimental import pallas as pl
from jax.experimental.pallas import tpu as pltpu

N_DEV = 4
SQ = 256
SKV_SHARD = 4096
HQ = 8
DH = 128
DM = HQ * DH
SCALE = 0.08838834764831843
BLK = 64
LCOLS = 16
PACK = DM + HQ * LCOLS


def kernel(x, Wq, K_ext, V_ext, Wo):
    x2 = x.reshape(SQ, DM)
    K2 = K_ext.reshape(SKV_SHARD, DM)
    V2 = V_ext.reshape(SKV_SHARD, DM)

    def body(x_ref, wq_ref, k_ref, v_ref, wo_ref, out_ref,
             acc_ref, ctx_ref, send_sems, recv_sems):
        my = lax.axis_index("i")
        left = (my + N_DEV - 1) % N_DEV
        right = (my + 1) % N_DEV

        barrier = pltpu.get_barrier_semaphore()
        for nbr in (left, right):
            pl.semaphore_signal(
                barrier, inc=1,
                device_id=(nbr,), device_id_type=pl.DeviceIdType.MESH,
            )
        pl.semaphore_wait(barrier, 2)

        q = jnp.dot(x_ref[...], wq_ref[...],
                    preferred_element_type=jnp.float32) * SCALE

        qb = lax.broadcasted_iota(jnp.int32, (SQ, SKV_SHARD), 0) // BLK
        kb = (lax.broadcasted_iota(jnp.int32, (SQ, SKV_SHARD), 1)
              + my * SKV_SHARD) // BLK
        mask = (qb == kb) | (kb == 0) | (((qb + kb) % 3) == 0)

        for h in range(HQ):
            q_h = q[:, h * DH:(h + 1) * DH]
            k_h = k_ref[:, h * DH:(h + 1) * DH]
            s = lax.dot_general(q_h, k_h, (((1,), (1,)), ((), ())),
                                preferred_element_type=jnp.float32)
            w = jnp.where(mask, jnp.exp(s), 0.0)
            l_h = jnp.sum(w, axis=1, keepdims=True)
            ctx_h = jnp.dot(w, v_ref[:, h * DH:(h + 1) * DH],
                            preferred_element_type=jnp.float32)
            acc_ref[0, :, h * DH:(h + 1) * DH] = ctx_h
            acc_ref[0, :, DM + h * LCOLS:DM + (h + 1) * LCOLS] = (
                jnp.broadcast_to(l_h, (SQ, LCOLS)))

        for hop in range(N_DEV - 1):
            rdma = pltpu.make_async_remote_copy(
                src_ref=acc_ref.at[hop],
                dst_ref=acc_ref.at[hop + 1],
                send_sem=send_sems.at[hop],
                recv_sem=recv_sems.at[hop],
                device_id=(right,),
                device_id_type=pl.DeviceIdType.MESH,
            )
            rdma.start()
            rdma.wait()

        total = acc_ref[0] + acc_ref[1] + acc_ref[2] + acc_ref[3]
        for h in range(HQ):
            l_h = total[:, DM + h * LCOLS:DM + h * LCOLS + 1]
            ctx_ref[:, h * DH:(h + 1) * DH] = (
                total[:, h * DH:(h + 1) * DH] / l_h)
        out_ref[...] = jnp.dot(ctx_ref[...], wo_ref[...],
                               preferred_element_type=jnp.float32)

    out = pl.pallas_call(
        body,
        out_shape=jax.ShapeDtypeStruct((SQ, DM), jnp.float32),
        in_specs=[pl.BlockSpec(memory_space=pltpu.VMEM)] * 5,
        out_specs=pl.BlockSpec(memory_space=pltpu.VMEM),
        scratch_shapes=[
            pltpu.VMEM((N_DEV, SQ, PACK), jnp.float32),
            pltpu.VMEM((SQ, DM), jnp.float32),
            pltpu.SemaphoreType.DMA((N_DEV - 1,)),
            pltpu.SemaphoreType.DMA((N_DEV - 1,)),
        ],
        compiler_params=pltpu.CompilerParams(collective_id=0),
    )(x2, Wq, K2, V2, Wo)
    return out.reshape(1, SQ, DM)


# baseline (device time: 106174 ns/iter reference)
import jax
import jax.numpy as jnp
from jax import lax
from jax.experimental import pallas as pl
from jax.experimental.pallas import tpu as pltpu

N_DEV = 4
SQ = 256
SKV_SHARD = 4096
HQ = 8
DH = 128
DM = HQ * DH
SCALE = 0.08838834764831843
BLK = 64
LCOLS = 16
PACK = DM + HQ * LCOLS


def kernel(x, Wq, K_ext, V_ext, Wo):
    x2 = x.reshape(SQ, DM)
    K2 = K_ext.reshape(SKV_SHARD, DM)
    V2 = V_ext.reshape(SKV_SHARD, DM)

    def body(x_ref, wq_ref, k_ref, v_ref, wo_ref, out_ref,
             acc_ref, ctx_ref, send_sems, recv_sems):
        my = lax.axis_index("i")
        left = (my + N_DEV - 1) % N_DEV
        right = (my + 1) % N_DEV

        barrier = pltpu.get_barrier_semaphore()
        for nbr in (left, right):
            pl.semaphore_signal(
                barrier, inc=1,
                device_id=(nbr,), device_id_type=pl.DeviceIdType.MESH,
            )
        pl.semaphore_wait(barrier, 2)

        q = jnp.dot(x_ref[...], wq_ref[...],
                    preferred_element_type=jnp.float32) * SCALE

        qb = lax.broadcasted_iota(jnp.int32, (SQ, SKV_SHARD), 0) // BLK
        kb = (lax.broadcasted_iota(jnp.int32, (SQ, SKV_SHARD), 1)
              + my * SKV_SHARD) // BLK
        mask = (qb == kb) | (kb == 0) | (((qb + kb) % 3) == 0)

        for h in range(HQ):
            q_h = q[:, h * DH:(h + 1) * DH]
            k_h = k_ref[:, h * DH:(h + 1) * DH]
            s = lax.dot_general(q_h, k_h, (((1,), (1,)), ((), ())),
                                preferred_element_type=jnp.float32)
            w = jnp.where(mask, jnp.exp(s), 0.0)
            l_h = jnp.sum(w, axis=1, keepdims=True)
            ctx_h = jnp.dot(w, v_ref[:, h * DH:(h + 1) * DH],
                            preferred_element_type=jnp.float32)
            acc_ref[0, :, h * DH:(h + 1) * DH] = ctx_h
            acc_ref[0, :, DM + h * LCOLS:DM + (h + 1) * LCOLS] = (
                jnp.broadcast_to(l_h, (SQ, LCOLS)))

        for hop in range(N_DEV - 1):
            rdma = pltpu.make_async_remote_copy(
                src_ref=acc_ref.at[hop],
                dst_ref=acc_ref.at[hop + 1],
                send_sem=send_sems.at[hop],
                recv_sem=recv_sems.at[hop],
                device_id=(right,),
                device_id_type=pl.DeviceIdType.MESH,
            )
            rdma.start()
            rdma.wait()

        total = acc_ref[0] + acc_ref[1] + acc_ref[2] + acc_ref[3]
        for h in range(HQ):
            l_h = total[:, DM + h * LCOLS:DM + h * LCOLS + 1]
            ctx_ref[:, h * DH:(h + 1) * DH] = (
                total[:, h * DH:(h + 1) * DH] / l_h)
        out_ref[...] = jnp.dot(ctx_ref[...], wo_ref[...],
                               preferred_element_type=jnp.float32)

    out = pl.pallas_call(
        body,
        out_shape=jax.ShapeDtypeStruct((SQ, DM), jnp.float32),
        in_specs=[pl.BlockSpec(memory_space=pltpu.VMEM)] * 5,
        out_specs=pl.BlockSpec(memory_space=pltpu.VMEM),
        scratch_shapes=[
            pltpu.VMEM((N_DEV, SQ, PACK), jnp.float32),
            pltpu.VMEM((SQ, DM), jnp.float32),
            pltpu.SemaphoreType.DMA((N_DEV - 1,)),
            pltpu.SemaphoreType.DMA((N_DEV - 1,)),
        ],
        compiler_params=pltpu.CompilerParams(
            collective_id=0,
            vmem_limit_bytes=100 * 1024 * 1024,
        ),
    )(x2, Wq, K2, V2, Wo)
    return out.reshape(1, SQ, DM)


# device time: 62545 ns/iter; 1.6976x vs baseline; 1.6976x over previous
import jax
import jax.numpy as jnp
from jax import lax
from jax.experimental import pallas as pl
from jax.experimental.pallas import tpu as pltpu

N_DEV = 4
_RING = True
SQ = 256
SKV_SHARD = 4096
HQ = 8
DH = 128
DM = HQ * DH
SCALE = 0.08838834764831843
BLK = 64
LCOLS = 16
PACK = DM + HQ * LCOLS


def kernel(x, Wq, K_ext, V_ext, Wo):
    x2 = x.reshape(SQ, DM)
    K2 = K_ext.reshape(SKV_SHARD, DM)
    V2 = V_ext.reshape(SKV_SHARD, DM)

    def body(x_ref, wq_ref, k_ref, v_ref, wo_ref, out_ref,
             acc_ref, ctx_ref, send_sems, recv_sems):
        my = lax.axis_index("i")
        left = (my + N_DEV - 1) % N_DEV
        right = (my + 1) % N_DEV

        barrier = pltpu.get_barrier_semaphore()
        for nbr in (left, right):
            pl.semaphore_signal(
                barrier, inc=1,
                device_id=(nbr,), device_id_type=pl.DeviceIdType.MESH,
            )
        pl.semaphore_wait(barrier, 2)

        q = jnp.dot(x_ref[...], wq_ref[...],
                    preferred_element_type=jnp.float32) * SCALE

        qb = lax.broadcasted_iota(jnp.int32, (SQ, SKV_SHARD), 0) // BLK
        kb = (lax.broadcasted_iota(jnp.int32, (SQ, SKV_SHARD), 1)
              + my * SKV_SHARD) // BLK
        mask = (qb == kb) | (kb == 0) | (((qb + kb) % 3) == 0)

        for h in range(HQ):
            q_h = q[:, h * DH:(h + 1) * DH]
            k_h = k_ref[:, h * DH:(h + 1) * DH]
            s = lax.dot_general(q_h, k_h, (((1,), (1,)), ((), ())),
                                preferred_element_type=jnp.float32)
            w = jnp.where(mask, jnp.exp(s), 0.0)
            l_h = jnp.sum(w, axis=1, keepdims=True)
            ctx_h = jnp.dot(w, v_ref[:, h * DH:(h + 1) * DH],
                            preferred_element_type=jnp.float32)
            acc_ref[0, :, h * DH:(h + 1) * DH] = ctx_h
            acc_ref[0, :, DM + h * LCOLS:DM + (h + 1) * LCOLS] = (
                jnp.broadcast_to(l_h, (SQ, LCOLS)))

        for hop in range(N_DEV - 1 if _RING else 0):
            rdma = pltpu.make_async_remote_copy(
                src_ref=acc_ref.at[hop],
                dst_ref=acc_ref.at[hop + 1],
                send_sem=send_sems.at[hop],
                recv_sem=recv_sems.at[hop],
                device_id=(right,),
                device_id_type=pl.DeviceIdType.MESH,
            )
            rdma.start()
            rdma.wait()

        total = acc_ref[0] + acc_ref[1] + acc_ref[2] + acc_ref[3]
        for h in range(HQ):
            l_h = total[:, DM + h * LCOLS:DM + h * LCOLS + 1]
            ctx_ref[:, h * DH:(h + 1) * DH] = (
                total[:, h * DH:(h + 1) * DH] / l_h)
        out_ref[...] = jnp.dot(ctx_ref[...], wo_ref[...],
                               preferred_element_type=jnp.float32)

    out = pl.pallas_call(
        body,
        out_shape=jax.ShapeDtypeStruct((SQ, DM), jnp.float32),
        in_specs=[pl.BlockSpec(memory_space=pltpu.VMEM)] * 5,
        out_specs=pl.BlockSpec(memory_space=pltpu.VMEM),
        scratch_shapes=[
            pltpu.VMEM((N_DEV, SQ, PACK), jnp.float32),
            pltpu.VMEM((SQ, DM), jnp.float32),
            pltpu.SemaphoreType.DMA((N_DEV - 1,)),
            pltpu.SemaphoreType.DMA((N_DEV - 1,)),
        ],
        compiler_params=pltpu.CompilerParams(
            collective_id=0,
            vmem_limit_bytes=100 * 1024 * 1024,
        ),
    )(x2, Wq, K2, V2, Wo)
    return out.reshape(1, SQ, DM)


# device time: 59805 ns/iter; 1.7753x vs baseline; 1.0458x over previous
import jax
import jax.numpy as jnp
from jax import lax
from jax.experimental import pallas as pl
from jax.experimental.pallas import tpu as pltpu

N_DEV = 4
SQ = 256
SKV_SHARD = 4096
HQ = 8
DH = 128
DM = HQ * DH
SCALE = 0.08838834764831843
BLK = 64
LCOLS = 16
PACK = DM + HQ * LCOLS


def kernel(x, Wq, K_ext, V_ext, Wo):
    x2 = x.reshape(SQ, DM)
    K3 = K_ext.reshape(SKV_SHARD, HQ, DH)
    V3 = V_ext.reshape(SKV_SHARD, HQ, DH)

    def body(x_ref, wq_ref, k_any, v_any, wo_ref, out_ref,
             k_buf, v_buf, slot_ref, recv_ref, ctx_ref,
             kv_sems, ex_send, ex_recv):
        my = lax.axis_index("i")
        p0 = my ^ 1
        p1 = 3 - my

        barrier = pltpu.get_barrier_semaphore()
        for nbr in (p0, p1):
            pl.semaphore_signal(
                barrier, inc=1,
                device_id=(nbr,), device_id_type=pl.DeviceIdType.MESH,
            )
        pl.semaphore_wait(barrier, 2)

        def kv_dma(h, slot):
            k = pltpu.make_async_copy(
                k_any.at[:, h, :], k_buf.at[slot], kv_sems.at[slot, 0])
            v = pltpu.make_async_copy(
                v_any.at[:, h, :], v_buf.at[slot], kv_sems.at[slot, 1])
            return k, v

        dmas = {}
        dmas[0] = kv_dma(0, 0)
        for d in dmas[0]:
            d.start()

        q = jnp.dot(x_ref[...], wq_ref[...],
                    preferred_element_type=jnp.float32) * SCALE

        qb = lax.broadcasted_iota(jnp.int32, (SQ, SKV_SHARD), 0) // BLK
        kb = (lax.broadcasted_iota(jnp.int32, (SQ, SKV_SHARD), 1)
              + my * SKV_SHARD) // BLK
        mask = (qb == kb) | (kb == 0) | (((qb + kb) % 3) == 0)

        for h in range(HQ):
            slot = h % 2
            if h + 1 < HQ:
                dmas[h + 1] = kv_dma(h + 1, (h + 1) % 2)
                for d in dmas[h + 1]:
                    d.start()
            for d in dmas[h]:
                d.wait()
            q_h = q[:, h * DH:(h + 1) * DH]
            s = lax.dot_general(q_h, k_buf[slot], (((1,), (1,)), ((), ())),
                                preferred_element_type=jnp.float32)
            w = jnp.where(mask, jnp.exp(s), 0.0)
            l_h = jnp.sum(w, axis=1, keepdims=True)
            ctx_h = jnp.dot(w, v_buf[slot],
                            preferred_element_type=jnp.float32)
            slot_ref[:, h * DH:(h + 1) * DH] = ctx_h
            slot_ref[:, DM + h * LCOLS:DM + (h + 1) * LCOLS] = (
                jnp.broadcast_to(l_h, (SQ, LCOLS)))

        for s, p in enumerate((p0, p1)):
            rdma = pltpu.make_async_remote_copy(
                src_ref=slot_ref,
                dst_ref=recv_ref.at[s],
                send_sem=ex_send.at[s],
                recv_sem=ex_recv.at[s],
                device_id=(p,),
                device_id_type=pl.DeviceIdType.MESH,
            )
            rdma.start()
            rdma.wait()
            slot_ref[...] = slot_ref[...] + recv_ref[s]

        for h in range(HQ):
            l_h = slot_ref[:, DM + h * LCOLS:DM + h * LCOLS + 1]
            ctx_ref[:, h * DH:(h + 1) * DH] = (
                slot_ref[:, h * DH:(h + 1) * DH] / l_h)
        out_ref[...] = jnp.dot(ctx_ref[...], wo_ref[...],
                               preferred_element_type=jnp.float32)

    out = pl.pallas_call(
        body,
        out_shape=jax.ShapeDtypeStruct((SQ, DM), jnp.float32),
        in_specs=[
            pl.BlockSpec(memory_space=pltpu.VMEM),
            pl.BlockSpec(memory_space=pltpu.VMEM),
            pl.BlockSpec(memory_space=pl.ANY),
            pl.BlockSpec(memory_space=pl.ANY),
            pl.BlockSpec(memory_space=pltpu.VMEM),
        ],
        out_specs=pl.BlockSpec(memory_space=pltpu.VMEM),
        scratch_shapes=[
            pltpu.VMEM((2, SKV_SHARD, DH), jnp.float32),
            pltpu.VMEM((2, SKV_SHARD, DH), jnp.float32),
            pltpu.VMEM((SQ, PACK), jnp.float32),
            pltpu.VMEM((2, SQ, PACK), jnp.float32),
            pltpu.VMEM((SQ, DM), jnp.float32),
            pltpu.SemaphoreType.DMA((2, 2)),
            pltpu.SemaphoreType.DMA((2,)),
            pltpu.SemaphoreType.DMA((2,)),
        ],
        compiler_params=pltpu.CompilerParams(
            collective_id=0,
            vmem_limit_bytes=100 * 1024 * 1024,
        ),
    )(x2, Wq, K3, V3, Wo)
    return out.reshape(1, SQ, DM)


# device time: 46258 ns/iter; 2.2953x vs baseline; 1.2929x over previous
import jax
import jax.numpy as jnp
from jax import lax
from jax.experimental import pallas as pl
from jax.experimental.pallas import tpu as pltpu

N_DEV = 4
SQ = 256
SKV_SHARD = 4096
HQ = 8
DH = 128
DM = HQ * DH
SCALE = 0.08838834764831843
BLK = 64
LCOLS = 16
PLANE = DH + LCOLS

LAG1 = 2
LAG2 = 4


def kernel(x, Wq, K_ext, V_ext, Wo):
    x2 = x.reshape(SQ, DM)
    K3 = K_ext.reshape(SKV_SHARD, HQ, DH)
    V3 = V_ext.reshape(SKV_SHARD, HQ, DH)

    def body(x_ref, wq_ref, k_any, v_any, wo_ref, out_ref,
             k_buf, v_buf, slot_ref, recv0_ref, recv1_ref,
             kv_sems, s0_send, s0_recv, s1_send, s1_recv):
        my = lax.axis_index("i")
        p0 = my ^ 1
        p1 = 3 - my

        barrier = pltpu.get_barrier_semaphore()
        for nbr in (p0, p1):
            pl.semaphore_signal(
                barrier, inc=1,
                device_id=(nbr,), device_id_type=pl.DeviceIdType.MESH,
            )
        pl.semaphore_wait(barrier, 2)

        def kv_dma(h, slot):
            k = pltpu.make_async_copy(
                k_any.at[:, h, :], k_buf.at[slot], kv_sems.at[slot, 0])
            v = pltpu.make_async_copy(
                v_any.at[:, h, :], v_buf.at[slot], kv_sems.at[slot, 1])
            return k, v

        def exchange(stage, h):
            src, dst, ssem, rsem, p = (
                (slot_ref, recv0_ref, s0_send, s0_recv, p0) if stage == 0
                else (slot_ref, recv1_ref, s1_send, s1_recv, p1))
            return pltpu.make_async_remote_copy(
                src_ref=src.at[h],
                dst_ref=dst.at[h],
                send_sem=ssem.at[h],
                recv_sem=rsem.at[h],
                device_id=(p,),
                device_id_type=pl.DeviceIdType.MESH,
            )

        kv_dmas = {0: kv_dma(0, 0)}
        for d in kv_dmas[0]:
            d.start()

        q = jnp.dot(x_ref[...], wq_ref[...],
                    preferred_element_type=jnp.float32) * SCALE

        qb = lax.broadcasted_iota(jnp.int32, (SQ, SKV_SHARD), 0) // BLK
        kb = (lax.broadcasted_iota(jnp.int32, (SQ, SKV_SHARD), 1)
              + my * SKV_SHARD) // BLK
        mask = (qb == kb) | (kb == 0) | (((qb + kb) % 3) == 0)

        s0 = {}
        s1 = {}

        def do_stage0_add_and_stage1(g):
            s0[g].wait()
            slot_ref[g] = slot_ref[g] + recv0_ref[g]
            s1[g] = exchange(1, g)
            s1[g].start()

        def do_stage1_add_and_project(f):
            s1[f].wait()
            total = slot_ref[f] + recv1_ref[f]
            ctx_n = total[:, :DH] / total[:, DH:DH + 1]
            term = jnp.dot(ctx_n, wo_ref[f * DH:(f + 1) * DH, :],
                           preferred_element_type=jnp.float32)
            if f == 0:
                out_ref[...] = term
            else:
                out_ref[...] = out_ref[...] + term

        for h in range(HQ):
            slot = h % 2
            if h + 1 < HQ:
                kv_dmas[h + 1] = kv_dma(h + 1, (h + 1) % 2)
                for d in kv_dmas[h + 1]:
                    d.start()
            for d in kv_dmas[h]:
                d.wait()
            q_h = q[:, h * DH:(h + 1) * DH]
            s = lax.dot_general(q_h, k_buf[slot], (((1,), (1,)), ((), ())),
                                preferred_element_type=jnp.float32)
            w = jnp.where(mask, jnp.exp(s), 0.0)
            l_h = jnp.sum(w, axis=1, keepdims=True)
            ctx_h = jnp.dot(w, v_buf[slot],
                            preferred_element_type=jnp.float32)
            slot_ref[h, :, :DH] = ctx_h
            slot_ref[h, :, DH:] = jnp.broadcast_to(l_h, (SQ, LCOLS))
            s0[h] = exchange(0, h)
            s0[h].start()
            if h >= LAG1:
                do_stage0_add_and_stage1(h - LAG1)
            if h >= LAG2:
                do_stage1_add_and_project(h - LAG2)

        for g in range(HQ - LAG1, HQ):
            do_stage0_add_and_stage1(g)
        for f in range(HQ - LAG2, HQ):
            do_stage1_add_and_project(f)

    out = pl.pallas_call(
        body,
        out_shape=jax.ShapeDtypeStruct((SQ, DM), jnp.float32),
        in_specs=[
            pl.BlockSpec(memory_space=pltpu.VMEM),
            pl.BlockSpec(memory_space=pltpu.VMEM),
            pl.BlockSpec(memory_space=pl.ANY),
            pl.BlockSpec(memory_space=pl.ANY),
            pl.BlockSpec(memory_space=pltpu.VMEM),
        ],
        out_specs=pl.BlockSpec(memory_space=pltpu.VMEM),
        scratch_shapes=[
            pltpu.VMEM((2, SKV_SHARD, DH), jnp.float32),
            pltpu.VMEM((2, SKV_SHARD, DH), jnp.float32),
            pltpu.VMEM((HQ, SQ, PLANE), jnp.float32),
            pltpu.VMEM((HQ, SQ, PLANE), jnp.float32),
            pltpu.VMEM((HQ, SQ, PLANE), jnp.float32),
            pltpu.SemaphoreType.DMA((2, 2)),
            pltpu.SemaphoreType.DMA((HQ,)),
            pltpu.SemaphoreType.DMA((HQ,)),
            pltpu.SemaphoreType.DMA((HQ,)),
            pltpu.SemaphoreType.DMA((HQ,)),
        ],
        compiler_params=pltpu.CompilerParams(
            collective_id=0,
            vmem_limit_bytes=100 * 1024 * 1024,
        ),
    )(x2, Wq, K3, V3, Wo)
    return out.reshape(1, SQ, DM)
